# baseline (speedup 1.0000x reference)
import functools

import jax
import jax.numpy as jnp
from jax import lax
from jax.experimental import pallas as pl
from jax.experimental.pallas import tpu as pltpu
from jax.experimental.pallas import tpu_sc as plsc

M = 262144
D = 64
B = 131072
H = 128

NC = 2
NS = 16
L = 16

CHUNK = 16384
NPASS = (M // NC) // CHUNK
SCAN = B // NS
ROWS_PT = CHUNK // NS
CH = 128
NTRASH = 16
CAP = SCAN + 2 * CH


def _mlp_body(val_ref, w1_ref, b1_ref, w2_ref, b2_ref, out_ref):
    h = jnp.dot(val_ref[...], w1_ref[...], preferred_element_type=jnp.float32)
    h = h + b1_ref[...]
    h = jnp.where(h >= 0, h, 0.25 * h)
    u = jnp.dot(h, w2_ref[...], preferred_element_type=jnp.float32)
    u = u + b2_ref[...]
    out_ref[...] = jnp.where(u >= 0, u, 0.25 * u)


def _mlp(val, W1, b1, W2, b2):
    blk = 8192
    grid = (B // blk,)
    return pl.pallas_call(
        _mlp_body,
        grid=grid,
        in_specs=[
            pl.BlockSpec((blk, D), lambda i: (i, 0)),
            pl.BlockSpec((D, H), lambda i: (0, 0)),
            pl.BlockSpec((1, H), lambda i: (0, 0)),
            pl.BlockSpec((H, D), lambda i: (0, 0)),
            pl.BlockSpec((1, D), lambda i: (0, 0)),
        ],
        out_specs=pl.BlockSpec((blk, D), lambda i: (i, 0)),
        out_shape=jax.ShapeDtypeStruct((B, D), jnp.float32),
    )(val, W1, b1.reshape(1, H), W2, b2.reshape(1, D))


def _sc_body(mem_hbm, idx_hbm, upd_hbm, out_hbm,
             idx_v, posf, rowf, pos2, row2, gbuf, shared, sem):
    c = lax.axis_index("c")
    s = lax.axis_index("s")

    pltpu.sync_copy(idx_hbm.at[pl.ds(s * SCAN, SCAN)], idx_v)

    lanes = lax.iota(jnp.int32, L)

    def one_pass(k, _):
        gbase = c * (M // NC) + k * CHUNK

        pltpu.sync_copy(mem_hbm.at[pl.ds(gbase + s * ROWS_PT, ROWS_PT)],
                        shared.at[pl.ds(s * ROWS_PT, ROWS_PT)])

        def scan_step(i, n):
            v = idx_v[pl.ds(i * L, L)]
            m = (v >= gbase) & (v < gbase + CHUNK)
            pos = s * SCAN + i * L + lanes
            pf = plsc.cumsum(m.astype(jnp.int32))
            off = n + pf - 1
            plsc.store_scatter(posf, [off], pos, mask=m)
            plsc.store_scatter(rowf, [off], v - gbase, mask=m)
            return n + pf[L - 1]


        n = lax.fori_loop(0, SCAN // L, scan_step, jnp.int32(0))

        a0 = (n // L) * L
        for t in range(CH // L + 1):
            a = a0 + t * L
            keep = (a + lanes) < n
            pv = posf[pl.ds(a, L)]
            rv = rowf[pl.ds(a, L)]
            posf[pl.ds(a, L)] = jnp.where(keep, pv, lanes)
            rowf[pl.ds(a, L)] = jnp.where(keep, rv, CHUNK + lanes)

        plsc.subcore_barrier()

        nch = (n + CH - 1) // CH

        def chunk_step(j, _):
            for t in range(CH // L):
                pos2[pl.ds(t * L, L)] = posf[pl.ds(j * CH + t * L, L)]
                row2[pl.ds(t * L, L)] = rowf[pl.ds(j * CH + t * L, L)]
            pltpu.async_copy(upd_hbm.at[pos2], gbuf, sem).wait()
            pltpu.sync_copy(gbuf, shared.at[row2], add=True)
            return 0

        lax.fori_loop(0, nch, chunk_step, 0)
        plsc.subcore_barrier()

        pltpu.sync_copy(shared.at[pl.ds(s * ROWS_PT, ROWS_PT)],
                        out_hbm.at[pl.ds(gbase + s * ROWS_PT, ROWS_PT)])
        return 0

    lax.fori_loop(0, NPASS, one_pass, 0)


def _scatter(mem, idx, upd):
    mesh = plsc.VectorSubcoreMesh(core_axis_name="c", subcore_axis_name="s")
    f = pl.kernel(
        _sc_body,
        out_type=jax.ShapeDtypeStruct((M, D), jnp.float32),
        mesh=mesh,
        compiler_params=pltpu.CompilerParams(
            needs_layout_passes=False, use_tc_tiling_on_sc=False),
        scratch_types=[
            pltpu.VMEM((SCAN,), jnp.int32),
            pltpu.VMEM((CAP,), jnp.int32),
            pltpu.VMEM((CAP,), jnp.int32),
            pltpu.VMEM((CH,), jnp.int32),
            pltpu.VMEM((CH,), jnp.int32),
            pltpu.VMEM((CH, D), jnp.float32),
            pltpu.VMEM_SHARED((CHUNK + NTRASH, D), jnp.float32),
            pltpu.SemaphoreType.DMA,
        ],
    )
    return f(mem, idx, upd)


def kernel(mem, idx, val, W1, b1, W2, b2):
    upd = _mlp(val, W1, b1, W2, b2)
    return _scatter(mem, idx, upd)
